# baseline (speedup 1.0000x reference)
import jax
import jax.numpy as jnp
from jax.experimental import pallas as pl
from jax.experimental.pallas import tpu as pltpu


def _dist_kernel(x_ref, w_ref, o_ref):
    x = x_ref[...]
    tb, s, l = x.shape
    x = jnp.log1p(jnp.abs(x))

    w = w_ref[...] * jnp.float32(0.5)
    p = jnp.tanh(jnp.dot(x.reshape(tb * s, l), w,
                         preferred_element_type=jnp.float32))
    p3 = p.reshape(tb, s, -1)

    g = jax.lax.dot_general(
        p3, p3, (((2,), (2,)), ((0,), (0,))),
        preferred_element_type=jnp.float32)

    gt = jnp.transpose(g.reshape(tb, s * s))
    gt3 = gt.reshape(s, s, tb)
    nt = 0.25 * jnp.concatenate(
        [gt[(s + 1) * i:(s + 1) * i + 1] for i in range(s)])
    d2 = (nt[:, None, :] + nt[None, :, :]) - 0.5 * gt3
    y = jnp.maximum(d2, 1e-37)
    o_ref[...] = y * jax.lax.rsqrt(y)


def kernel(sentences, weights):
    B, S, L = sentences.shape
    Lw, R = weights.shape
    assert L == Lw

    TB = 128
    NB = pl.cdiv(B, TB)
    B_pad = NB * TB
    if B_pad != B:
        sentences = jnp.pad(sentences, ((0, B_pad - B), (0, 0), (0, 0)))

    out = pl.pallas_call(
        _dist_kernel,
        out_shape=jax.ShapeDtypeStruct((S, S, B_pad), jnp.float32),
        grid=(NB,),
        in_specs=[
            pl.BlockSpec((TB, S, L), lambda b: (b, 0, 0)),
            pl.BlockSpec((L, R), lambda b: (0, 0)),
        ],
        out_specs=pl.BlockSpec((S, S, TB), lambda b: (0, 0, b)),
        compiler_params=pltpu.CompilerParams(
            dimension_semantics=("parallel",)),
        cost_estimate=pl.CostEstimate(
            flops=2 * B_pad * S * L * R + 2 * B_pad * S * S * R,
            transcendentals=B_pad * S * (L + R + S),
            bytes_accessed=4 * (B_pad * S * L + L * R + B_pad * S * S)),
    )(sentences, weights)

    out = jnp.transpose(out, (2, 0, 1))
    return out[:B] if B_pad != B else out


# speedup vs baseline: 1.0808x; 1.0808x over previous
import jax
import jax.numpy as jnp
from jax.experimental import pallas as pl
from jax.experimental.pallas import tpu as pltpu


def _dist_kernel(x_ref, w_ref, o_ref):
    x = x_ref[...]
    tb, s, l = x.shape
    x = jnp.log2(jnp.abs(x) + 1.0)

    w = w_ref[...] * jnp.float32(0.5 * 0.6931471805599453)
    p = jnp.tanh(jnp.dot(x.reshape(tb * s, l), w,
                         preferred_element_type=jnp.float32))
    p3 = p.reshape(tb, s, -1)

    g = jax.lax.dot_general(
        p3, p3, (((2,), (2,)), ((0,), (0,))),
        preferred_element_type=jnp.float32)

    gt = jnp.transpose(g.reshape(tb, s * s))
    gt3 = gt.reshape(s, s, tb)
    nt = 0.25 * jnp.concatenate(
        [gt[(s + 1) * i:(s + 1) * i + 1] for i in range(s)])
    d2 = (nt[:, None, :] + nt[None, :, :]) - 0.5 * gt3
    y = jnp.maximum(d2, 1e-37)
    o_ref[...] = y * jax.lax.rsqrt(y)


def kernel(sentences, weights):
    B, S, L = sentences.shape
    Lw, R = weights.shape
    assert L == Lw

    TB = 128
    NB = pl.cdiv(B, TB)
    B_pad = NB * TB
    if B_pad != B:
        sentences = jnp.pad(sentences, ((0, B_pad - B), (0, 0), (0, 0)))

    out = pl.pallas_call(
        _dist_kernel,
        out_shape=jax.ShapeDtypeStruct((S, S, B_pad), jnp.float32),
        grid=(NB,),
        in_specs=[
            pl.BlockSpec((TB, S, L), lambda b: (b, 0, 0)),
            pl.BlockSpec((L, R), lambda b: (0, 0)),
        ],
        out_specs=pl.BlockSpec((S, S, TB), lambda b: (0, 0, b)),
        compiler_params=pltpu.CompilerParams(
            dimension_semantics=("parallel",)),
        cost_estimate=pl.CostEstimate(
            flops=2 * B_pad * S * L * R + 2 * B_pad * S * S * R,
            transcendentals=B_pad * S * (L + R + S),
            bytes_accessed=4 * (B_pad * S * L + L * R + B_pad * S * S)),
    )(sentences, weights)

    out = jnp.transpose(out, (2, 0, 1))
    return out[:B] if B_pad != B else out
